# baseline (speedup 1.0000x reference)
import functools

import jax
import jax.numpy as jnp
from jax import lax
from jax.experimental import pallas as pl
from jax.experimental.pallas import tpu as pltpu
from jax.experimental.pallas import tpu_sc as plsc

N_ROWS = 1000000
DIM = 64
BATCH = 16384
MAX_NORM = 2.0
BAND = 256
SHIFT = 8
N_BAND = (N_ROWS + BAND - 1) // BAND
LAST_BB = 7813 * 128 - BAND

NC = 2
NS = 16
L = 16
NW = NC * NS
B_PER_W = BATCH // NW
LIST_CAP = 2064
NSLOT = 32
NSUB = 16
SUBSHIFT = 3
CAP_SUB = 128
NBM = 128
CH2 = 256


def _rsqrt_newton(x):
    i = lax.bitcast_convert_type(x, jnp.int32)
    i = jnp.int32(0x5F3759DF) - lax.shift_right_arithmetic(i, 1)
    y = lax.bitcast_convert_type(i, jnp.float32)
    xh = x * 0.5
    for _ in range(3):
        y = y * (1.5 - xh * y * y)
    return y


def _splat(x):
    return jnp.full((L,), 0, jnp.int32) + x


def _extract_body(uid_hbm, iid_hbm, utab_hbm, itab_hbm, uscr_hbm, iscr_hbm,
                  uid_v, iid_v, ulr, ule, ilr, ile, uslr, usle, marks,
                  hitbands, rb, stage_v,
                  sem_b0, sem_b1, sem_b2, sem_b3, sem_w):
    wid = lax.axis_index("s") * NC + lax.axis_index("c")
    b_lo = lax.shift_right_logical(wid * N_BAND, 5)
    b_hi = lax.shift_right_logical((wid + 1) * N_BAND, 5)

    pltpu.sync_copy(uid_hbm, uid_v)
    pltpu.sync_copy(iid_hbm, iid_v)

    ids16 = lax.iota(jnp.int32, L)
    lo_s = _splat(b_lo)
    hi_s = _splat(b_hi)
    ones_i = jnp.full((L,), 1, jnp.int32)
    sems = [sem_b0, sem_b1, sem_b2, sem_b3]

    def bin_body(k, carry):
        off_u, off_i = carry
        e16 = k * L + ids16
        ru = uid_v[pl.ds(k * L, L)]
        bu = lax.shift_right_logical(ru, SHIFT)
        mu = jnp.logical_and(bu >= lo_s, bu < hi_s)
        plsc.store_compressed(ulr.at[pl.ds(off_u, L)], ru, mask=mu)
        plsc.store_compressed(ule.at[pl.ds(off_u, L)], e16, mask=mu)
        pcu = plsc.all_reduce_population_count(mu)[0]
        ri = iid_v[pl.ds(k * L, L)]
        bi = lax.shift_right_logical(ri, SHIFT)
        mi = jnp.logical_and(bi >= lo_s, bi < hi_s)
        plsc.store_compressed(ilr.at[pl.ds(off_i, L)], ri, mask=mi)
        plsc.store_compressed(ile.at[pl.ds(off_i, L)], e16, mask=mi)
        pci = plsc.all_reduce_population_count(mi)[0]
        return (off_u + pcu, off_i + pci)

    nloc_u, nloc_i = lax.fori_loop(0, BATCH // L, bin_body, (0, 0),
                                   unroll=2)

    def band_phase(lr_ref, le_ref, slr, sle, nloc, tab_hbm, scr_hbm, slot0):
        nvec = lax.shift_right_logical(nloc + L - 1, 4)
        nloc_s = _splat(nloc)
        b_lo_s = _splat(b_lo)

        sub_offs = []
        for subr in range(NSUB):
            s_s = _splat(subr)

            def sub_body(v, off, s_s=s_s, subr=subr):
                lane = v * L + ids16
                lr = lr_ref[pl.ds(v * L, L)]
                le = le_ref[pl.ds(v * L, L)]
                sub = lax.shift_right_logical(
                    lax.shift_right_logical(lr, SHIFT) - b_lo_s, SUBSHIFT)
                m = jnp.logical_and(sub == s_s, lane < nloc_s)
                dst = pl.ds(subr * CAP_SUB + off, L)
                plsc.store_compressed(slr.at[dst], lr, mask=m)
                plsc.store_compressed(sle.at[dst], le, mask=m)
                return off + plsc.all_reduce_population_count(m)[0]

            sub_offs.append(lax.fori_loop(0, nvec, sub_body, 0))
        counts_vec = jnp.zeros((L,), jnp.int32)
        for subr in range(NSUB):
            counts_vec = jnp.where(ids16 == subr, _splat(sub_offs[subr]),
                                   counts_vec)

        def zero_body(v, _):
            marks[pl.ds(v * L, L)] = jnp.zeros((L,), jnp.int32)
            return 0

        lax.fori_loop(0, NBM // L, zero_body, 0)

        def mark_body(v, _):
            lane = v * L + ids16
            lr = lr_ref[pl.ds(v * L, L)]
            bl = lax.shift_right_logical(lr, SHIFT) - b_lo_s
            m = lane < nloc_s
            plsc.store_scatter(marks, [bl], ones_i, mask=m)
            return 0

        lax.fori_loop(0, nvec, mark_body, 0)

        def comp_body(v, off):
            bv = v * L + ids16
            mk = marks[pl.ds(v * L, L)]
            m = mk == ones_i
            plsc.store_compressed(hitbands.at[pl.ds(off, L)], bv + b_lo_s,
                                  mask=m)
            return off + plsc.all_reduce_population_count(m)[0]

        nhit = lax.fori_loop(0, NBM // L, comp_body, 0)

        def hb_at(j):
            g = plsc.load_gather(hitbands, [_splat(j)])
            return jnp.max(g)

        def fire(j):
            b = hb_at(j)
            buf = jnp.bitwise_and(j, 3)
            bb = pl.multiple_of(jnp.minimum(b * BAND, LAST_BB), 128)
            for q in range(4):
                @pl.when(buf == q)
                def _(q=q):
                    pltpu.async_copy(
                        tab_hbm.at[pl.ds(0, DIM), pl.ds(bb, BAND)],
                        rb.at[q], sems[q])

        lax.fori_loop(0, jnp.minimum(4, nhit),
                      lambda k, _: (fire(k), 0)[1], 0)

        def band_body(j, slot):
            b = hb_at(j)
            buf = jnp.bitwise_and(j, 3)
            for q in range(4):
                @pl.when(buf == q)
                def _(q=q):
                    pltpu.make_async_copy(
                        tab_hbm.at[pl.ds(0, DIM), pl.ds(0, BAND)],
                        rb.at[q], sems[q]).wait()

            b_s = _splat(b)
            buf_s = _splat(buf)
            sub = lax.shift_right_logical(b - b_lo, SUBSHIFT)
            base_s = sub * CAP_SUB
            cnt_sub = jnp.max(jnp.where(ids16 == _splat(sub), counts_vec, 0))
            cnt_s = _splat(cnt_sub)
            nvec_sub = lax.shift_right_logical(cnt_sub + L - 1, 4)

            def vec_body(v, slot):
                lane = v * L + ids16
                sl = pl.ds(base_s + v * L, L)
                lr = slr[sl]
                le = sle[sl]
                m = jnp.logical_and(
                    lax.shift_right_logical(lr, SHIFT) == b_s, lane < cnt_s)
                pc = plsc.all_reduce_population_count(m)[0]

                def hit_body(h, carry):
                    mrem, slot = carry
                    lidx = plsc.all_reduce_ffs(mrem)
                    onehot = ids16 == lidx
                    bb_s = jnp.minimum(b * BAND, LAST_BB)
                    rr = jnp.max(jnp.where(onehot, lr, 0)) - bb_s
                    e_s = jnp.max(jnp.where(onehot, le, 0))
                    rr16 = _splat(rr)
                    for jb in range(4):
                        g = plsc.load_gather(
                            rb, [buf_s, ids16 + jb * L, rr16])
                        stage_v[slot, pl.ds(jb * L, L)] = g
                    pltpu.async_copy(stage_v.at[pl.ds(slot, 1)],
                                     scr_hbm.at[pl.ds(e_s, 1)], sem_w)
                    return (jnp.logical_and(mrem,
                                            jnp.logical_not(onehot)),
                            jnp.bitwise_and(slot + 1, NSLOT - 1))

                _, slot = lax.fori_loop(0, pc, hit_body, (m, slot))
                return slot

            slot = lax.fori_loop(0, nvec_sub, vec_body, slot)

            @pl.when(j + 4 < nhit)
            def _():
                fire(j + 4)
            return slot

        slot = lax.fori_loop(0, nhit, band_body, slot0)

        def drain(k, _):
            pltpu.make_async_copy(stage_v.at[pl.ds(0, 1)],
                                  scr_hbm.at[pl.ds(0, 1)], sem_w).wait()
            return 0

        lax.fori_loop(0, nloc, drain, 0)
        return slot

    slot = band_phase(ulr, ule, uslr, usle, nloc_u, utab_hbm, uscr_hbm, 0)
    band_phase(ilr, ile, uslr, usle, nloc_i, itab_hbm, iscr_hbm, slot)


def _compute_body(uid_hbm, iid_hbm, uscr_hbm, iscr_hbm, out_hbm,
                  uidx_v, iidx_v, urows_v, irows_v, out_v, sem_u, sem_i):
    wid = lax.axis_index("s") * NC + lax.axis_index("c")
    base = wid * B_PER_W

    pltpu.sync_copy(uid_hbm.at[pl.ds(base, B_PER_W)], uidx_v)
    pltpu.sync_copy(iid_hbm.at[pl.ds(base, B_PER_W)], iidx_v)

    ids16 = lax.iota(jnp.int32, L)
    zeros = jnp.zeros((L,), jnp.float32)

    def chunk_body(c, _):
        cb = c * CH2
        cu = pltpu.async_copy(uscr_hbm.at[pl.ds(base + cb, CH2)], urows_v,
                              sem_u)
        ci = pltpu.async_copy(iscr_hbm.at[pl.ds(base + cb, CH2)], irows_v,
                              sem_i)
        cu.wait()
        ci.wait()

        def group_body(g, _):
            e16 = g * L + ids16

            def feat_body(j, carry):
                uu, vv, uv = carry
                j16 = _splat(j)
                u = plsc.load_gather(urows_v, [e16, j16])
                v = plsc.load_gather(irows_v, [e16, j16])
                return (uu + u * u, vv + v * v, uv + u * v)

            uu, vv, uv = lax.fori_loop(0, DIM, feat_body,
                                       (zeros, zeros, zeros), unroll=True)

            su = jnp.where(uu > MAX_NORM * MAX_NORM,
                           MAX_NORM * _rsqrt_newton(uu), 1.0)
            sv = jnp.where(vv > MAX_NORM * MAX_NORM,
                           MAX_NORM * _rsqrt_newton(vv), 1.0)
            dot = su * sv * uv
            rating = 5.0 / (1.0 + jnp.exp(-dot))
            plsc.store_scatter(out_v, [cb + e16], rating)
            return 0

        lax.fori_loop(0, CH2 // L, group_body, 0)
        return 0

    lax.fori_loop(0, B_PER_W // CH2, chunk_body, 0)

    pltpu.sync_copy(out_v, out_hbm.at[pl.ds(base, B_PER_W)])


@jax.jit
def kernel(user_id, item_id, users_table, items_table):
    utabT = users_table.T
    itabT = items_table.T
    mesh = plsc.VectorSubcoreMesh(core_axis_name="c", subcore_axis_name="s")
    params = pltpu.CompilerParams(needs_layout_passes=False)

    extract = functools.partial(
        pl.kernel,
        out_type=(jax.ShapeDtypeStruct((BATCH, DIM), jnp.float32),
                  jax.ShapeDtypeStruct((BATCH, DIM), jnp.float32)),
        mesh=mesh,
        compiler_params=params,
        scratch_types=[
            pltpu.VMEM((BATCH,), jnp.int32),
            pltpu.VMEM((BATCH,), jnp.int32),
            pltpu.VMEM((LIST_CAP,), jnp.int32),
            pltpu.VMEM((LIST_CAP,), jnp.int32),
            pltpu.VMEM((LIST_CAP,), jnp.int32),
            pltpu.VMEM((LIST_CAP,), jnp.int32),
            pltpu.VMEM((NSUB * CAP_SUB + L,), jnp.int32),
            pltpu.VMEM((NSUB * CAP_SUB + L,), jnp.int32),
            pltpu.VMEM((NBM,), jnp.int32),
            pltpu.VMEM((NBM + L,), jnp.int32),
            pltpu.VMEM((4, DIM, BAND), jnp.float32),
            pltpu.VMEM((NSLOT, DIM), jnp.float32),
            pltpu.SemaphoreType.DMA,
            pltpu.SemaphoreType.DMA,
            pltpu.SemaphoreType.DMA,
            pltpu.SemaphoreType.DMA,
            pltpu.SemaphoreType.DMA,
        ],
    )(_extract_body)
    uscr, iscr = extract(user_id, item_id, utabT, itabT)

    compute = functools.partial(
        pl.kernel,
        out_type=jax.ShapeDtypeStruct((BATCH,), jnp.float32),
        mesh=mesh,
        compiler_params=params,
        scratch_types=[
            pltpu.VMEM((B_PER_W,), jnp.int32),
            pltpu.VMEM((B_PER_W,), jnp.int32),
            pltpu.VMEM((CH2, DIM), jnp.float32),
            pltpu.VMEM((CH2, DIM), jnp.float32),
            pltpu.VMEM((B_PER_W,), jnp.float32),
            pltpu.SemaphoreType.DMA,
            pltpu.SemaphoreType.DMA,
        ],
    )(_compute_body)
    return compute(user_id, item_id, uscr, iscr)


# speedup vs baseline: 1.1069x; 1.1069x over previous
import functools

import jax
import jax.numpy as jnp
from jax import lax
from jax.experimental import pallas as pl
from jax.experimental.pallas import tpu as pltpu
from jax.experimental.pallas import tpu_sc as plsc

N_ROWS = 1000000
DIM = 64
BATCH = 16384
MAX_NORM = 2.0
BAND = 128
SHIFT = 7
N_BAND = (N_ROWS + BAND - 1) // BAND

NC = 2
NS = 16
L = 16
NW = NC * NS
B_PER_W = BATCH // NW
LIST_CAP = 2064
NSLOT = 32
NSUB = 16
SUBSHIFT = 4
CAP_SUB = 128
NBM = 256
CH2 = 256


def _rsqrt_newton(x):
    i = lax.bitcast_convert_type(x, jnp.int32)
    i = jnp.int32(0x5F3759DF) - lax.shift_right_arithmetic(i, 1)
    y = lax.bitcast_convert_type(i, jnp.float32)
    xh = x * 0.5
    for _ in range(3):
        y = y * (1.5 - xh * y * y)
    return y


def _splat(x):
    return jnp.full((L,), 0, jnp.int32) + x


def _extract_body(uid_hbm, iid_hbm, utab_hbm, itab_hbm, uscr_hbm, iscr_hbm,
                  uid_v, iid_v, ulr, ule, ilr, ile, uslr, usle, marks,
                  hitbands, rb, stage_v,
                  sem_b0, sem_b1, sem_b2, sem_b3, sem_b4, sem_b5, sem_b6,
                  sem_b7, sem_w):
    wid = lax.axis_index("s") * NC + lax.axis_index("c")
    b_lo = lax.shift_right_logical(wid * N_BAND, 5)
    b_hi = lax.shift_right_logical((wid + 1) * N_BAND, 5)

    pltpu.sync_copy(uid_hbm, uid_v)
    pltpu.sync_copy(iid_hbm, iid_v)

    ids16 = lax.iota(jnp.int32, L)
    lo_s = _splat(b_lo)
    hi_s = _splat(b_hi)
    ones_i = jnp.full((L,), 1, jnp.int32)
    sems = [sem_b0, sem_b1, sem_b2, sem_b3, sem_b4, sem_b5, sem_b6, sem_b7]

    def bin_body(k, carry):
        off_u, off_i = carry
        e16 = k * L + ids16
        ru = uid_v[pl.ds(k * L, L)]
        bu = lax.shift_right_logical(ru, SHIFT)
        mu = jnp.logical_and(bu >= lo_s, bu < hi_s)
        plsc.store_compressed(ulr.at[pl.ds(off_u, L)], ru, mask=mu)
        plsc.store_compressed(ule.at[pl.ds(off_u, L)], e16, mask=mu)
        pcu = plsc.all_reduce_population_count(mu)[0]
        ri = iid_v[pl.ds(k * L, L)]
        bi = lax.shift_right_logical(ri, SHIFT)
        mi = jnp.logical_and(bi >= lo_s, bi < hi_s)
        plsc.store_compressed(ilr.at[pl.ds(off_i, L)], ri, mask=mi)
        plsc.store_compressed(ile.at[pl.ds(off_i, L)], e16, mask=mi)
        pci = plsc.all_reduce_population_count(mi)[0]
        return (off_u + pcu, off_i + pci)

    nloc_u, nloc_i = lax.fori_loop(0, BATCH // L, bin_body, (0, 0),
                                   unroll=2)

    def band_phase(lr_ref, le_ref, slr, sle, nloc, tab_hbm, scr_hbm, slot0):
        nvec = lax.shift_right_logical(nloc + L - 1, 4)
        nloc_s = _splat(nloc)
        b_lo_s = _splat(b_lo)

        sub_offs = []
        for subr in range(NSUB):
            s_s = _splat(subr)

            def sub_body(v, off, s_s=s_s, subr=subr):
                lane = v * L + ids16
                lr = lr_ref[pl.ds(v * L, L)]
                le = le_ref[pl.ds(v * L, L)]
                sub = lax.shift_right_logical(
                    lax.shift_right_logical(lr, SHIFT) - b_lo_s, SUBSHIFT)
                m = jnp.logical_and(sub == s_s, lane < nloc_s)
                dst = pl.ds(subr * CAP_SUB + off, L)
                plsc.store_compressed(slr.at[dst], lr, mask=m)
                plsc.store_compressed(sle.at[dst], le, mask=m)
                return off + plsc.all_reduce_population_count(m)[0]

            sub_offs.append(lax.fori_loop(0, nvec, sub_body, 0))
        counts_vec = jnp.zeros((L,), jnp.int32)
        for subr in range(NSUB):
            counts_vec = jnp.where(ids16 == subr, _splat(sub_offs[subr]),
                                   counts_vec)

        def zero_body(v, _):
            marks[pl.ds(v * L, L)] = jnp.zeros((L,), jnp.int32)
            return 0

        lax.fori_loop(0, NBM // L, zero_body, 0)

        def mark_body(v, _):
            lane = v * L + ids16
            lr = lr_ref[pl.ds(v * L, L)]
            bl = lax.shift_right_logical(lr, SHIFT) - b_lo_s
            m = lane < nloc_s
            plsc.store_scatter(marks, [bl], ones_i, mask=m)
            return 0

        lax.fori_loop(0, nvec, mark_body, 0)

        def comp_body(v, off):
            bv = v * L + ids16
            mk = marks[pl.ds(v * L, L)]
            m = mk == ones_i
            plsc.store_compressed(hitbands.at[pl.ds(off, L)], bv + b_lo_s,
                                  mask=m)
            return off + plsc.all_reduce_population_count(m)[0]

        nhit = lax.fori_loop(0, NBM // L, comp_body, 0)

        def hb_at(j):
            g = plsc.load_gather(hitbands, [_splat(j)])
            return jnp.max(g)

        def fire(j):
            b = hb_at(j)
            buf = jnp.bitwise_and(j, 7)
            bb = pl.multiple_of(b * BAND, BAND)
            for q in range(8):
                @pl.when(buf == q)
                def _(q=q):
                    pltpu.async_copy(
                        tab_hbm.at[pl.ds(0, DIM), pl.ds(bb, BAND)],
                        rb.at[q], sems[q])

        lax.fori_loop(0, jnp.minimum(8, nhit),
                      lambda k, _: (fire(k), 0)[1], 0)

        def band_body(j, slot):
            b = hb_at(j)
            buf = jnp.bitwise_and(j, 7)
            for q in range(8):
                @pl.when(buf == q)
                def _(q=q):
                    pltpu.make_async_copy(
                        tab_hbm.at[pl.ds(0, DIM), pl.ds(0, BAND)],
                        rb.at[q], sems[q]).wait()

            b_s = _splat(b)
            buf_s = _splat(buf)
            sub = lax.shift_right_logical(b - b_lo, SUBSHIFT)
            base_s = sub * CAP_SUB
            cnt_sub = jnp.max(jnp.where(ids16 == _splat(sub), counts_vec, 0))
            cnt_s = _splat(cnt_sub)
            nvec_sub = lax.shift_right_logical(cnt_sub + L - 1, 4)

            def vec_body(v, slot):
                lane = v * L + ids16
                sl = pl.ds(base_s + v * L, L)
                lr = slr[sl]
                le = sle[sl]
                m = jnp.logical_and(
                    lax.shift_right_logical(lr, SHIFT) == b_s, lane < cnt_s)
                pc = plsc.all_reduce_population_count(m)[0]

                def hit_body(h, carry):
                    mrem, slot = carry
                    lidx = plsc.all_reduce_ffs(mrem)
                    onehot = ids16 == lidx
                    rr = jnp.max(jnp.where(onehot,
                                           jnp.bitwise_and(lr, BAND - 1), 0))
                    e_s = jnp.max(jnp.where(onehot, le, 0))
                    rr16 = _splat(rr)
                    for jb in range(4):
                        g = plsc.load_gather(
                            rb, [buf_s, ids16 + jb * L, rr16])
                        stage_v[slot, pl.ds(jb * L, L)] = g
                    pltpu.async_copy(stage_v.at[pl.ds(slot, 1)],
                                     scr_hbm.at[pl.ds(e_s, 1)], sem_w)
                    return (jnp.logical_and(mrem,
                                            jnp.logical_not(onehot)),
                            jnp.bitwise_and(slot + 1, NSLOT - 1))

                _, slot = lax.fori_loop(0, pc, hit_body, (m, slot))
                return slot

            slot = lax.fori_loop(0, nvec_sub, vec_body, slot)

            @pl.when(j + 8 < nhit)
            def _():
                fire(j + 8)
            return slot

        slot = lax.fori_loop(0, nhit, band_body, slot0)

        def drain(k, _):
            pltpu.make_async_copy(stage_v.at[pl.ds(0, 1)],
                                  scr_hbm.at[pl.ds(0, 1)], sem_w).wait()
            return 0

        lax.fori_loop(0, nloc, drain, 0)
        return slot

    slot = band_phase(ulr, ule, uslr, usle, nloc_u, utab_hbm, uscr_hbm, 0)
    band_phase(ilr, ile, uslr, usle, nloc_i, itab_hbm, iscr_hbm, slot)


def _compute_body(uid_hbm, iid_hbm, uscr_hbm, iscr_hbm, out_hbm,
                  uidx_v, iidx_v, urows_v, irows_v, out_v, sem_u, sem_i):
    wid = lax.axis_index("s") * NC + lax.axis_index("c")
    base = wid * B_PER_W

    pltpu.sync_copy(uid_hbm.at[pl.ds(base, B_PER_W)], uidx_v)
    pltpu.sync_copy(iid_hbm.at[pl.ds(base, B_PER_W)], iidx_v)

    ids16 = lax.iota(jnp.int32, L)
    zeros = jnp.zeros((L,), jnp.float32)

    def chunk_body(c, _):
        cb = c * CH2
        cu = pltpu.async_copy(uscr_hbm.at[pl.ds(base + cb, CH2)], urows_v,
                              sem_u)
        ci = pltpu.async_copy(iscr_hbm.at[pl.ds(base + cb, CH2)], irows_v,
                              sem_i)
        cu.wait()
        ci.wait()

        def group_body(g, _):
            e16 = g * L + ids16

            def feat_body(j, carry):
                uu, vv, uv = carry
                j16 = _splat(j)
                u = plsc.load_gather(urows_v, [e16, j16])
                v = plsc.load_gather(irows_v, [e16, j16])
                return (uu + u * u, vv + v * v, uv + u * v)

            uu, vv, uv = lax.fori_loop(0, DIM, feat_body,
                                       (zeros, zeros, zeros), unroll=True)

            su = jnp.where(uu > MAX_NORM * MAX_NORM,
                           MAX_NORM * _rsqrt_newton(uu), 1.0)
            sv = jnp.where(vv > MAX_NORM * MAX_NORM,
                           MAX_NORM * _rsqrt_newton(vv), 1.0)
            dot = su * sv * uv
            rating = 5.0 / (1.0 + jnp.exp(-dot))
            plsc.store_scatter(out_v, [cb + e16], rating)
            return 0

        lax.fori_loop(0, CH2 // L, group_body, 0)
        return 0

    lax.fori_loop(0, B_PER_W // CH2, chunk_body, 0)

    pltpu.sync_copy(out_v, out_hbm.at[pl.ds(base, B_PER_W)])


@jax.jit
def kernel(user_id, item_id, users_table, items_table):
    utabT = users_table.T
    itabT = items_table.T
    mesh = plsc.VectorSubcoreMesh(core_axis_name="c", subcore_axis_name="s")
    params = pltpu.CompilerParams(needs_layout_passes=False)

    extract = functools.partial(
        pl.kernel,
        out_type=(jax.ShapeDtypeStruct((BATCH, DIM), jnp.float32),
                  jax.ShapeDtypeStruct((BATCH, DIM), jnp.float32)),
        mesh=mesh,
        compiler_params=params,
        scratch_types=[
            pltpu.VMEM((BATCH,), jnp.int32),
            pltpu.VMEM((BATCH,), jnp.int32),
            pltpu.VMEM((LIST_CAP,), jnp.int32),
            pltpu.VMEM((LIST_CAP,), jnp.int32),
            pltpu.VMEM((LIST_CAP,), jnp.int32),
            pltpu.VMEM((LIST_CAP,), jnp.int32),
            pltpu.VMEM((NSUB * CAP_SUB + L,), jnp.int32),
            pltpu.VMEM((NSUB * CAP_SUB + L,), jnp.int32),
            pltpu.VMEM((NBM,), jnp.int32),
            pltpu.VMEM((NBM + L,), jnp.int32),
            pltpu.VMEM((8, DIM, BAND), jnp.float32),
            pltpu.VMEM((NSLOT, DIM), jnp.float32),
            pltpu.SemaphoreType.DMA,
            pltpu.SemaphoreType.DMA,
            pltpu.SemaphoreType.DMA,
            pltpu.SemaphoreType.DMA,
            pltpu.SemaphoreType.DMA,
            pltpu.SemaphoreType.DMA,
            pltpu.SemaphoreType.DMA,
            pltpu.SemaphoreType.DMA,
            pltpu.SemaphoreType.DMA,
        ],
    )(_extract_body)
    uscr, iscr = extract(user_id, item_id, utabT, itabT)

    compute = functools.partial(
        pl.kernel,
        out_type=jax.ShapeDtypeStruct((BATCH,), jnp.float32),
        mesh=mesh,
        compiler_params=params,
        scratch_types=[
            pltpu.VMEM((B_PER_W,), jnp.int32),
            pltpu.VMEM((B_PER_W,), jnp.int32),
            pltpu.VMEM((CH2, DIM), jnp.float32),
            pltpu.VMEM((CH2, DIM), jnp.float32),
            pltpu.VMEM((B_PER_W,), jnp.float32),
            pltpu.SemaphoreType.DMA,
            pltpu.SemaphoreType.DMA,
        ],
    )(_compute_body)
    return compute(user_id, item_id, uscr, iscr)


# speedup vs baseline: 1.1087x; 1.0016x over previous
import functools

import jax
import jax.numpy as jnp
from jax import lax
from jax.experimental import pallas as pl
from jax.experimental.pallas import tpu as pltpu
from jax.experimental.pallas import tpu_sc as plsc

N_ROWS = 1000000
DIM = 64
BATCH = 16384
MAX_NORM = 2.0
BAND = 128
SHIFT = 7
N_BAND = (N_ROWS + BAND - 1) // BAND

NC = 2
NS = 16
L = 16
NW = NC * NS
B_PER_W = BATCH // NW
LIST_CAP = 2064
NSLOT = 32
NSUB = 16
SUBSHIFT = 4
CAP_SUB = 128
NBM = 256
CH2 = 256


def _rsqrt_newton(x):
    i = lax.bitcast_convert_type(x, jnp.int32)
    i = jnp.int32(0x5F3759DF) - lax.shift_right_arithmetic(i, 1)
    y = lax.bitcast_convert_type(i, jnp.float32)
    xh = x * 0.5
    for _ in range(3):
        y = y * (1.5 - xh * y * y)
    return y


def _splat(x):
    return jnp.full((L,), 0, jnp.int32) + x


def _extract_body(uid_hbm, iid_hbm, utab_hbm, itab_hbm, uscr_hbm, iscr_hbm,
                  uid_v, iid_v, ulr, ule, ilr, ile, uslr, usle, marks,
                  hitbands, rb, stage_v,
                  sem_b0, sem_b1, sem_b2, sem_b3, sem_b4, sem_b5, sem_b6,
                  sem_b7, sem_w):
    wid = lax.axis_index("s") * NC + lax.axis_index("c")
    b_lo = lax.shift_right_logical(wid * N_BAND, 5)
    b_hi = lax.shift_right_logical((wid + 1) * N_BAND, 5)

    pltpu.sync_copy(uid_hbm, uid_v)
    pltpu.sync_copy(iid_hbm, iid_v)

    ids16 = lax.iota(jnp.int32, L)
    lo_s = _splat(b_lo)
    hi_s = _splat(b_hi)
    ones_i = jnp.full((L,), 1, jnp.int32)
    sems = [sem_b0, sem_b1, sem_b2, sem_b3, sem_b4, sem_b5, sem_b6, sem_b7]

    def bin_body(k, carry):
        off_u, off_i = carry
        e16 = k * L + ids16
        ru = uid_v[pl.ds(k * L, L)]
        bu = lax.shift_right_logical(ru, SHIFT)
        mu = jnp.logical_and(bu >= lo_s, bu < hi_s)
        plsc.store_compressed(ulr.at[pl.ds(off_u, L)], ru, mask=mu)
        plsc.store_compressed(ule.at[pl.ds(off_u, L)], e16, mask=mu)
        pcu = plsc.all_reduce_population_count(mu)[0]
        ri = iid_v[pl.ds(k * L, L)]
        bi = lax.shift_right_logical(ri, SHIFT)
        mi = jnp.logical_and(bi >= lo_s, bi < hi_s)
        plsc.store_compressed(ilr.at[pl.ds(off_i, L)], ri, mask=mi)
        plsc.store_compressed(ile.at[pl.ds(off_i, L)], e16, mask=mi)
        pci = plsc.all_reduce_population_count(mi)[0]
        return (off_u + pcu, off_i + pci)

    nloc_u, nloc_i = lax.fori_loop(0, BATCH // L, bin_body, (0, 0),
                                   unroll=2)

    def band_phase(lr_ref, le_ref, slr, sle, nloc, tab_hbm, scr_hbm, slot0):
        nvec = lax.shift_right_logical(nloc + L - 1, 4)
        nloc_s = _splat(nloc)
        b_lo_s = _splat(b_lo)

        sub_offs = []
        for subr in range(NSUB):
            s_s = _splat(subr)

            def sub_body(v, off, s_s=s_s, subr=subr):
                lane = v * L + ids16
                lr = lr_ref[pl.ds(v * L, L)]
                le = le_ref[pl.ds(v * L, L)]
                sub = lax.shift_right_logical(
                    lax.shift_right_logical(lr, SHIFT) - b_lo_s, SUBSHIFT)
                m = jnp.logical_and(sub == s_s, lane < nloc_s)
                dst = pl.ds(subr * CAP_SUB + off, L)
                plsc.store_compressed(slr.at[dst], lr, mask=m)
                plsc.store_compressed(sle.at[dst], le, mask=m)
                return off + plsc.all_reduce_population_count(m)[0]

            sub_offs.append(lax.fori_loop(0, nvec, sub_body, 0))
        counts_vec = jnp.zeros((L,), jnp.int32)
        for subr in range(NSUB):
            counts_vec = jnp.where(ids16 == subr, _splat(sub_offs[subr]),
                                   counts_vec)

        def zero_body(v, _):
            marks[pl.ds(v * L, L)] = jnp.zeros((L,), jnp.int32)
            return 0

        lax.fori_loop(0, NBM // L, zero_body, 0)

        def mark_body(v, _):
            lane = v * L + ids16
            lr = lr_ref[pl.ds(v * L, L)]
            bl = lax.shift_right_logical(lr, SHIFT) - b_lo_s
            m = lane < nloc_s
            plsc.store_scatter(marks, [bl], ones_i, mask=m)
            return 0

        lax.fori_loop(0, nvec, mark_body, 0)

        def comp_body(v, off):
            bv = v * L + ids16
            mk = marks[pl.ds(v * L, L)]
            m = mk == ones_i
            plsc.store_compressed(hitbands.at[pl.ds(off, L)], bv + b_lo_s,
                                  mask=m)
            return off + plsc.all_reduce_population_count(m)[0]

        nhit = lax.fori_loop(0, NBM // L, comp_body, 0)

        def hb_at(j):
            g = plsc.load_gather(hitbands, [_splat(j)])
            return jnp.max(g)

        def fire(j):
            b = hb_at(j)
            buf = jnp.bitwise_and(j, 7)
            bb = pl.multiple_of(b * BAND, BAND)
            for q in range(8):
                @pl.when(buf == q)
                def _(q=q):
                    pltpu.async_copy(
                        tab_hbm.at[pl.ds(0, DIM // 2), pl.ds(bb, BAND)],
                        rb.at[q, pl.ds(0, DIM // 2)], sems[q])
                    pltpu.async_copy(
                        tab_hbm.at[pl.ds(DIM // 2, DIM // 2),
                                   pl.ds(bb, BAND)],
                        rb.at[q, pl.ds(DIM // 2, DIM // 2)], sems[q])

        lax.fori_loop(0, jnp.minimum(8, nhit),
                      lambda k, _: (fire(k), 0)[1], 0)

        def band_body(j, slot):
            b = hb_at(j)
            buf = jnp.bitwise_and(j, 7)
            for q in range(8):
                @pl.when(buf == q)
                def _(q=q):
                    pltpu.make_async_copy(
                        tab_hbm.at[pl.ds(0, DIM // 2), pl.ds(0, BAND)],
                        rb.at[q, pl.ds(0, DIM // 2)], sems[q]).wait()
                    pltpu.make_async_copy(
                        tab_hbm.at[pl.ds(0, DIM // 2), pl.ds(0, BAND)],
                        rb.at[q, pl.ds(0, DIM // 2)], sems[q]).wait()

            b_s = _splat(b)
            buf_s = _splat(buf)
            sub = lax.shift_right_logical(b - b_lo, SUBSHIFT)
            base_s = sub * CAP_SUB
            cnt_sub = jnp.max(jnp.where(ids16 == _splat(sub), counts_vec, 0))
            cnt_s = _splat(cnt_sub)
            nvec_sub = lax.shift_right_logical(cnt_sub + L - 1, 4)

            def vec_body(v, slot):
                lane = v * L + ids16
                sl = pl.ds(base_s + v * L, L)
                lr = slr[sl]
                le = sle[sl]
                m = jnp.logical_and(
                    lax.shift_right_logical(lr, SHIFT) == b_s, lane < cnt_s)
                pc = plsc.all_reduce_population_count(m)[0]

                def hit_body(h, carry):
                    mrem, slot = carry
                    lidx = plsc.all_reduce_ffs(mrem)
                    onehot = ids16 == lidx
                    rr = jnp.max(jnp.where(onehot,
                                           jnp.bitwise_and(lr, BAND - 1), 0))
                    e_s = jnp.max(jnp.where(onehot, le, 0))
                    rr16 = _splat(rr)
                    for jb in range(4):
                        g = plsc.load_gather(
                            rb, [buf_s, ids16 + jb * L, rr16])
                        stage_v[slot, pl.ds(jb * L, L)] = g
                    pltpu.async_copy(stage_v.at[pl.ds(slot, 1)],
                                     scr_hbm.at[pl.ds(e_s, 1)], sem_w)
                    return (jnp.logical_and(mrem,
                                            jnp.logical_not(onehot)),
                            jnp.bitwise_and(slot + 1, NSLOT - 1))

                _, slot = lax.fori_loop(0, pc, hit_body, (m, slot))
                return slot

            slot = lax.fori_loop(0, nvec_sub, vec_body, slot)

            @pl.when(j + 8 < nhit)
            def _():
                fire(j + 8)
            return slot

        slot = lax.fori_loop(0, nhit, band_body, slot0)

        def drain(k, _):
            pltpu.make_async_copy(stage_v.at[pl.ds(0, 1)],
                                  scr_hbm.at[pl.ds(0, 1)], sem_w).wait()
            return 0

        lax.fori_loop(0, nloc, drain, 0)
        return slot

    slot = band_phase(ulr, ule, uslr, usle, nloc_u, utab_hbm, uscr_hbm, 0)
    band_phase(ilr, ile, uslr, usle, nloc_i, itab_hbm, iscr_hbm, slot)


def _compute_body(uid_hbm, iid_hbm, uscr_hbm, iscr_hbm, out_hbm,
                  uidx_v, iidx_v, urows_v, irows_v, out_v, sem_u, sem_i):
    wid = lax.axis_index("s") * NC + lax.axis_index("c")
    base = wid * B_PER_W

    pltpu.sync_copy(uid_hbm.at[pl.ds(base, B_PER_W)], uidx_v)
    pltpu.sync_copy(iid_hbm.at[pl.ds(base, B_PER_W)], iidx_v)

    ids16 = lax.iota(jnp.int32, L)
    zeros = jnp.zeros((L,), jnp.float32)

    def chunk_body(c, _):
        cb = c * CH2
        cu = pltpu.async_copy(uscr_hbm.at[pl.ds(base + cb, CH2)], urows_v,
                              sem_u)
        ci = pltpu.async_copy(iscr_hbm.at[pl.ds(base + cb, CH2)], irows_v,
                              sem_i)
        cu.wait()
        ci.wait()

        def group_body(g, _):
            e16 = g * L + ids16

            def feat_body(j, carry):
                uu, vv, uv = carry
                j16 = _splat(j)
                u = plsc.load_gather(urows_v, [e16, j16])
                v = plsc.load_gather(irows_v, [e16, j16])
                return (uu + u * u, vv + v * v, uv + u * v)

            uu, vv, uv = lax.fori_loop(0, DIM, feat_body,
                                       (zeros, zeros, zeros), unroll=True)

            su = jnp.where(uu > MAX_NORM * MAX_NORM,
                           MAX_NORM * _rsqrt_newton(uu), 1.0)
            sv = jnp.where(vv > MAX_NORM * MAX_NORM,
                           MAX_NORM * _rsqrt_newton(vv), 1.0)
            dot = su * sv * uv
            rating = 5.0 / (1.0 + jnp.exp(-dot))
            plsc.store_scatter(out_v, [cb + e16], rating)
            return 0

        lax.fori_loop(0, CH2 // L, group_body, 0)
        return 0

    lax.fori_loop(0, B_PER_W // CH2, chunk_body, 0)

    pltpu.sync_copy(out_v, out_hbm.at[pl.ds(base, B_PER_W)])


@jax.jit
def kernel(user_id, item_id, users_table, items_table):
    utabT = users_table.T
    itabT = items_table.T
    mesh = plsc.VectorSubcoreMesh(core_axis_name="c", subcore_axis_name="s")
    params = pltpu.CompilerParams(needs_layout_passes=False)

    extract = functools.partial(
        pl.kernel,
        out_type=(jax.ShapeDtypeStruct((BATCH, DIM), jnp.float32),
                  jax.ShapeDtypeStruct((BATCH, DIM), jnp.float32)),
        mesh=mesh,
        compiler_params=params,
        scratch_types=[
            pltpu.VMEM((BATCH,), jnp.int32),
            pltpu.VMEM((BATCH,), jnp.int32),
            pltpu.VMEM((LIST_CAP,), jnp.int32),
            pltpu.VMEM((LIST_CAP,), jnp.int32),
            pltpu.VMEM((LIST_CAP,), jnp.int32),
            pltpu.VMEM((LIST_CAP,), jnp.int32),
            pltpu.VMEM((NSUB * CAP_SUB + L,), jnp.int32),
            pltpu.VMEM((NSUB * CAP_SUB + L,), jnp.int32),
            pltpu.VMEM((NBM,), jnp.int32),
            pltpu.VMEM((NBM + L,), jnp.int32),
            pltpu.VMEM((8, DIM, BAND), jnp.float32),
            pltpu.VMEM((NSLOT, DIM), jnp.float32),
            pltpu.SemaphoreType.DMA,
            pltpu.SemaphoreType.DMA,
            pltpu.SemaphoreType.DMA,
            pltpu.SemaphoreType.DMA,
            pltpu.SemaphoreType.DMA,
            pltpu.SemaphoreType.DMA,
            pltpu.SemaphoreType.DMA,
            pltpu.SemaphoreType.DMA,
            pltpu.SemaphoreType.DMA,
        ],
    )(_extract_body)
    uscr, iscr = extract(user_id, item_id, utabT, itabT)

    compute = functools.partial(
        pl.kernel,
        out_type=jax.ShapeDtypeStruct((BATCH,), jnp.float32),
        mesh=mesh,
        compiler_params=params,
        scratch_types=[
            pltpu.VMEM((B_PER_W,), jnp.int32),
            pltpu.VMEM((B_PER_W,), jnp.int32),
            pltpu.VMEM((CH2, DIM), jnp.float32),
            pltpu.VMEM((CH2, DIM), jnp.float32),
            pltpu.VMEM((B_PER_W,), jnp.float32),
            pltpu.SemaphoreType.DMA,
            pltpu.SemaphoreType.DMA,
        ],
    )(_compute_body)
    return compute(user_id, item_id, uscr, iscr)


# speedup vs baseline: 1.1096x; 1.0008x over previous
import functools

import jax
import jax.numpy as jnp
from jax import lax
from jax.experimental import pallas as pl
from jax.experimental.pallas import tpu as pltpu
from jax.experimental.pallas import tpu_sc as plsc

N_ROWS = 1000000
DIM = 64
BATCH = 16384
MAX_NORM = 2.0
BAND = 128
SHIFT = 7
N_BAND = (N_ROWS + BAND - 1) // BAND

NC = 2
NS = 16
L = 16
NW = NC * NS
B_PER_W = BATCH // NW
LIST_CAP = 2064
NSLOT = 32
NSUB = 16
SUBSHIFT = 4
CAP_SUB = 128
NBM = 256
CH2 = 256


def _rsqrt_newton(x):
    i = lax.bitcast_convert_type(x, jnp.int32)
    i = jnp.int32(0x5F3759DF) - lax.shift_right_arithmetic(i, 1)
    y = lax.bitcast_convert_type(i, jnp.float32)
    xh = x * 0.5
    for _ in range(3):
        y = y * (1.5 - xh * y * y)
    return y


def _splat(x):
    return jnp.full((L,), 0, jnp.int32) + x


def _extract_body(uid_hbm, iid_hbm, utab_hbm, itab_hbm, uscr_hbm, iscr_hbm,
                  uid_v, iid_v, ulr, ule, ilr, ile, uslr, usle, marks,
                  hitbands, rb, stage_v,
                  sem_b0, sem_b1, sem_b2, sem_b3, sem_b4, sem_b5, sem_b6,
                  sem_b7, sem_w):
    wid = lax.axis_index("s") * NC + lax.axis_index("c")
    b_lo = lax.shift_right_logical(wid * N_BAND, 5)
    b_hi = lax.shift_right_logical((wid + 1) * N_BAND, 5)

    pltpu.sync_copy(uid_hbm, uid_v)
    pltpu.sync_copy(iid_hbm, iid_v)

    ids16 = lax.iota(jnp.int32, L)
    lo_s = _splat(b_lo)
    hi_s = _splat(b_hi)
    ones_i = jnp.full((L,), 1, jnp.int32)
    sems = [sem_b0, sem_b1, sem_b2, sem_b3, sem_b4, sem_b5, sem_b6, sem_b7]

    def bin_body(k, carry):
        off_u, off_i = carry
        e16 = k * L + ids16
        ru = uid_v[pl.ds(k * L, L)]
        bu = lax.shift_right_logical(ru, SHIFT)
        mu = jnp.logical_and(bu >= lo_s, bu < hi_s)
        plsc.store_compressed(ulr.at[pl.ds(off_u, L)], ru, mask=mu)
        plsc.store_compressed(ule.at[pl.ds(off_u, L)], e16, mask=mu)
        pcu = plsc.all_reduce_population_count(mu)[0]
        ri = iid_v[pl.ds(k * L, L)]
        bi = lax.shift_right_logical(ri, SHIFT)
        mi = jnp.logical_and(bi >= lo_s, bi < hi_s)
        plsc.store_compressed(ilr.at[pl.ds(off_i, L)], ri, mask=mi)
        plsc.store_compressed(ile.at[pl.ds(off_i, L)], e16, mask=mi)
        pci = plsc.all_reduce_population_count(mi)[0]
        return (off_u + pcu, off_i + pci)

    nloc_u, nloc_i = lax.fori_loop(0, BATCH // L, bin_body, (0, 0),
                                   unroll=2)

    def band_phase(lr_ref, le_ref, slr, sle, nloc, tab_hbm, scr_hbm, slot0):
        nvec = lax.shift_right_logical(nloc + L - 1, 4)
        nloc_s = _splat(nloc)
        b_lo_s = _splat(b_lo)

        sub_offs = []
        for subr in range(NSUB):
            s_s = _splat(subr)

            def sub_body(v, off, s_s=s_s, subr=subr):
                lane = v * L + ids16
                lr = lr_ref[pl.ds(v * L, L)]
                le = le_ref[pl.ds(v * L, L)]
                sub = lax.shift_right_logical(
                    lax.shift_right_logical(lr, SHIFT) - b_lo_s, SUBSHIFT)
                m = jnp.logical_and(sub == s_s, lane < nloc_s)
                dst = pl.ds(subr * CAP_SUB + off, L)
                plsc.store_compressed(slr.at[dst], lr, mask=m)
                plsc.store_compressed(sle.at[dst], le, mask=m)
                return off + plsc.all_reduce_population_count(m)[0]

            sub_offs.append(lax.fori_loop(0, nvec, sub_body, 0))
        counts_vec = jnp.zeros((L,), jnp.int32)
        for subr in range(NSUB):
            counts_vec = jnp.where(ids16 == subr, _splat(sub_offs[subr]),
                                   counts_vec)

        def zero_body(v, _):
            marks[pl.ds(v * L, L)] = jnp.zeros((L,), jnp.int32)
            return 0

        lax.fori_loop(0, NBM // L, zero_body, 0)

        def mark_body(v, _):
            lane = v * L + ids16
            lr = lr_ref[pl.ds(v * L, L)]
            bl = lax.shift_right_logical(lr, SHIFT) - b_lo_s
            m = lane < nloc_s
            plsc.store_scatter(marks, [bl], ones_i, mask=m)
            return 0

        lax.fori_loop(0, nvec, mark_body, 0)

        def comp_body(v, off):
            bv = v * L + ids16
            mk = marks[pl.ds(v * L, L)]
            m = mk == ones_i
            plsc.store_compressed(hitbands.at[pl.ds(off, L)], bv + b_lo_s,
                                  mask=m)
            return off + plsc.all_reduce_population_count(m)[0]

        nhit = lax.fori_loop(0, NBM // L, comp_body, 0)

        def hb_at(j):
            g = plsc.load_gather(hitbands, [_splat(j)])
            return jnp.max(g)

        def fire(j):
            b = hb_at(j)
            buf = jnp.bitwise_and(j, 7)
            bb = pl.multiple_of(b * BAND, BAND)
            for q in range(8):
                @pl.when(buf == q)
                def _(q=q):
                    pltpu.async_copy(
                        tab_hbm.at[pl.ds(0, DIM), pl.ds(bb, BAND)],
                        rb.at[q], sems[q])

        lax.fori_loop(0, jnp.minimum(8, nhit),
                      lambda k, _: (fire(k), 0)[1], 0)

        def band_body(j, slot):
            b = hb_at(j)
            buf = jnp.bitwise_and(j, 7)
            for q in range(8):
                @pl.when(buf == q)
                def _(q=q):
                    pltpu.make_async_copy(
                        tab_hbm.at[pl.ds(0, DIM), pl.ds(0, BAND)],
                        rb.at[q], sems[q]).wait()

            b_s = _splat(b)
            buf_s = _splat(buf)
            sub = lax.shift_right_logical(b - b_lo, SUBSHIFT)
            base_s = sub * CAP_SUB
            cnt_sub = jnp.max(jnp.where(ids16 == _splat(sub), counts_vec, 0))
            cnt_s = _splat(cnt_sub)
            nvec_sub = lax.shift_right_logical(cnt_sub + L - 1, 4)

            def vec_body(v, slot):
                lane = v * L + ids16
                sl = pl.ds(base_s + v * L, L)
                lr = slr[sl]
                le = sle[sl]
                m = jnp.logical_and(
                    lax.shift_right_logical(lr, SHIFT) == b_s, lane < cnt_s)
                pc = plsc.all_reduce_population_count(m)[0]

                def hit_body(h, carry):
                    mrem, slot = carry
                    lidx = plsc.all_reduce_ffs(mrem)
                    onehot = ids16 == lidx
                    rr = jnp.max(jnp.where(onehot,
                                           jnp.bitwise_and(lr, BAND - 1), 0))
                    e_s = jnp.max(jnp.where(onehot, le, 0))
                    rr16 = _splat(rr)
                    for jb in range(4):
                        g = plsc.load_gather(
                            rb, [buf_s, ids16 + jb * L, rr16])
                        stage_v[slot, pl.ds(jb * L, L)] = g
                    pltpu.async_copy(stage_v.at[pl.ds(slot, 1)],
                                     scr_hbm.at[pl.ds(e_s, 1)], sem_w)
                    return (jnp.logical_and(mrem,
                                            jnp.logical_not(onehot)),
                            jnp.bitwise_and(slot + 1, NSLOT - 1))

                _, slot = lax.fori_loop(0, pc, hit_body, (m, slot))
                return slot

            slot = lax.fori_loop(0, nvec_sub, vec_body, slot)

            @pl.when(j + 8 < nhit)
            def _():
                fire(j + 8)
            return slot

        slot = lax.fori_loop(0, nhit, band_body, slot0)

        def drain(k, _):
            pltpu.make_async_copy(stage_v.at[pl.ds(0, 1)],
                                  scr_hbm.at[pl.ds(0, 1)], sem_w).wait()
            return 0

        lax.fori_loop(0, nloc, drain, 0)
        return slot

    slot = band_phase(ulr, ule, uslr, usle, nloc_u, utab_hbm, uscr_hbm, 0)
    band_phase(ilr, ile, uslr, usle, nloc_i, itab_hbm, iscr_hbm, slot)


def _compute_body(uid_hbm, iid_hbm, uscr_hbm, iscr_hbm, out_hbm,
                  uidx_v, iidx_v, urows_v, irows_v, out_v, sem_u, sem_i):
    wid = lax.axis_index("s") * NC + lax.axis_index("c")
    base = wid * B_PER_W

    pltpu.sync_copy(uid_hbm.at[pl.ds(base, B_PER_W)], uidx_v)
    pltpu.sync_copy(iid_hbm.at[pl.ds(base, B_PER_W)], iidx_v)

    ids16 = lax.iota(jnp.int32, L)
    zeros = jnp.zeros((L,), jnp.float32)

    def chunk_body(c, _):
        cb = c * CH2
        cu = pltpu.async_copy(uscr_hbm.at[pl.ds(base + cb, CH2)], urows_v,
                              sem_u)
        ci = pltpu.async_copy(iscr_hbm.at[pl.ds(base + cb, CH2)], irows_v,
                              sem_i)
        cu.wait()
        ci.wait()

        def group_body(g, _):
            e16 = g * L + ids16

            def feat_body(j, carry):
                uu, vv, uv = carry
                j16 = _splat(j)
                u = plsc.load_gather(urows_v, [e16, j16])
                v = plsc.load_gather(irows_v, [e16, j16])
                return (uu + u * u, vv + v * v, uv + u * v)

            uu, vv, uv = lax.fori_loop(0, DIM, feat_body,
                                       (zeros, zeros, zeros), unroll=True)

            su = jnp.where(uu > MAX_NORM * MAX_NORM,
                           MAX_NORM * _rsqrt_newton(uu), 1.0)
            sv = jnp.where(vv > MAX_NORM * MAX_NORM,
                           MAX_NORM * _rsqrt_newton(vv), 1.0)
            dot = su * sv * uv
            rating = 5.0 / (1.0 + jnp.exp(-dot))
            plsc.store_scatter(out_v, [cb + e16], rating)
            return 0

        lax.fori_loop(0, CH2 // L, group_body, 0)
        return 0

    lax.fori_loop(0, B_PER_W // CH2, chunk_body, 0)

    pltpu.sync_copy(out_v, out_hbm.at[pl.ds(base, B_PER_W)])


@jax.jit
def kernel(user_id, item_id, users_table, items_table):
    utabT = users_table.T
    itabT = items_table.T
    mesh = plsc.VectorSubcoreMesh(core_axis_name="c", subcore_axis_name="s")
    params = pltpu.CompilerParams(needs_layout_passes=False)

    extract = functools.partial(
        pl.kernel,
        out_type=(jax.ShapeDtypeStruct((BATCH, DIM), jnp.float32),
                  jax.ShapeDtypeStruct((BATCH, DIM), jnp.float32)),
        mesh=mesh,
        compiler_params=params,
        scratch_types=[
            pltpu.VMEM((BATCH,), jnp.int32),
            pltpu.VMEM((BATCH,), jnp.int32),
            pltpu.VMEM((LIST_CAP,), jnp.int32),
            pltpu.VMEM((LIST_CAP,), jnp.int32),
            pltpu.VMEM((LIST_CAP,), jnp.int32),
            pltpu.VMEM((LIST_CAP,), jnp.int32),
            pltpu.VMEM((NSUB * CAP_SUB + L,), jnp.int32),
            pltpu.VMEM((NSUB * CAP_SUB + L,), jnp.int32),
            pltpu.VMEM((NBM,), jnp.int32),
            pltpu.VMEM((NBM + L,), jnp.int32),
            pltpu.VMEM((8, DIM, BAND), jnp.float32),
            pltpu.VMEM((NSLOT, DIM), jnp.float32),
            pltpu.SemaphoreType.DMA,
            pltpu.SemaphoreType.DMA,
            pltpu.SemaphoreType.DMA,
            pltpu.SemaphoreType.DMA,
            pltpu.SemaphoreType.DMA,
            pltpu.SemaphoreType.DMA,
            pltpu.SemaphoreType.DMA,
            pltpu.SemaphoreType.DMA,
            pltpu.SemaphoreType.DMA,
        ],
    )(_extract_body)
    uscr, iscr = extract(user_id, item_id, utabT, itabT)

    compute = functools.partial(
        pl.kernel,
        out_type=jax.ShapeDtypeStruct((BATCH,), jnp.float32),
        mesh=mesh,
        compiler_params=params,
        scratch_types=[
            pltpu.VMEM((B_PER_W,), jnp.int32),
            pltpu.VMEM((B_PER_W,), jnp.int32),
            pltpu.VMEM((CH2, DIM), jnp.float32),
            pltpu.VMEM((CH2, DIM), jnp.float32),
            pltpu.VMEM((B_PER_W,), jnp.float32),
            pltpu.SemaphoreType.DMA,
            pltpu.SemaphoreType.DMA,
        ],
    )(_compute_body)
    return compute(user_id, item_id, uscr, iscr)
